# baseline (speedup 1.0000x reference)
import jax
import jax.numpy as jnp
from jax.experimental import pallas as pl
from jax.experimental.pallas import tpu as pltpu

_EPS = 1e-12
_M = 1.7
_PHALF = -1.0 / (_M - 1.0)

_BN = 8192
_CH = 1024


def _fcm_body(x_ref, cs_ref, c2_ref, o_ref):
    cs = cs_ref[...]
    c2 = c2_ref[...]

    def step(i, carry):
        x = x_ref[pl.ds(i * _CH, _CH), :]
        x2 = jnp.sum(x * x, axis=1, keepdims=True)
        xc = jax.lax.dot_general(
            x, cs, (((1,), (1,)), ((), ())),
            preferred_element_type=jnp.float32)
        sq = x2 + (c2 + xc)
        t = jnp.maximum(sq, _EPS * _EPS)
        u = jnp.exp(_PHALF * jnp.log(t))
        o_ref[pl.ds(i * _CH, _CH), :] = u / jnp.sum(u, axis=1, keepdims=True)
        return carry

    jax.lax.fori_loop(0, _BN // _CH, step, 0)


def kernel(x, centers):
    N, F = x.shape
    K = centers.shape[0]
    cs = centers * (-2.0)
    c2 = jnp.sum(centers * centers, axis=1)[None, :]
    return pl.pallas_call(
        _fcm_body,
        grid=(N // _BN,),
        in_specs=[
            pl.BlockSpec((_BN, F), lambda i: (i, 0)),
            pl.BlockSpec((K, F), lambda i: (0, 0)),
            pl.BlockSpec((1, K), lambda i: (0, 0)),
        ],
        out_specs=pl.BlockSpec((_BN, K), lambda i: (i, 0)),
        out_shape=jax.ShapeDtypeStruct((N, K), jnp.float32),
        compiler_params=pltpu.CompilerParams(
            dimension_semantics=("parallel",)),
    )(x, cs, c2)


# speedup vs baseline: 1.2422x; 1.2422x over previous
import jax
import jax.numpy as jnp
from jax.experimental import pallas as pl
from jax.experimental.pallas import tpu as pltpu

_EPS = 1e-12
_M = 1.7
_PHALF = -1.0 / (_M - 1.0)

_BN = 8192
_CH = 1024


def _fcm_body(x_ref, cs_ref, c2_ref, o_ref):
    cs = cs_ref[...]
    c2 = c2_ref[...]
    x = x_ref[...]
    x2 = jnp.sum(x * x, axis=1, keepdims=True)
    xc = jax.lax.dot_general(
        x, cs, (((1,), (1,)), ((), ())),
        preferred_element_type=jnp.float32)
    sq = x2 + (c2 + xc)
    t = jnp.maximum(sq, _EPS * _EPS)
    u = jnp.exp(_PHALF * jnp.log(t))
    o_ref[...] = u / jnp.sum(u, axis=1, keepdims=True)


def kernel(x, centers):
    N, F = x.shape
    K = centers.shape[0]
    cs = centers * (-2.0)
    c2 = jnp.sum(centers * centers, axis=1)[None, :]
    return pl.pallas_call(
        _fcm_body,
        grid=(N // _BN,),
        in_specs=[
            pl.BlockSpec((_BN, F), lambda i: (i, 0)),
            pl.BlockSpec((K, F), lambda i: (0, 0)),
            pl.BlockSpec((1, K), lambda i: (0, 0)),
        ],
        out_specs=pl.BlockSpec((_BN, K), lambda i: (i, 0)),
        out_shape=jax.ShapeDtypeStruct((N, K), jnp.float32),
        compiler_params=pltpu.CompilerParams(
            dimension_semantics=("parallel",)),
    )(x, cs, c2)


# speedup vs baseline: 1.3087x; 1.0535x over previous
import jax
import jax.numpy as jnp
from jax.experimental import pallas as pl
from jax.experimental.pallas import tpu as pltpu

_EPS = 1e-12
_M = 1.7
_PHALF = -1.0 / (_M - 1.0)

_BN = 16384
_CH = 4096


def _fcm_body(x_ref, cs_ref, c2_ref, o_ref):
    cs = cs_ref[...]
    c2 = c2_ref[...]

    def step(i, carry):
        x = x_ref[pl.ds(i * _CH, _CH), :]
        x2 = jnp.sum(x * x, axis=1, keepdims=True)
        xc = jax.lax.dot_general(
            x, cs, (((1,), (1,)), ((), ())),
            preferred_element_type=jnp.float32)
        sq = x2 + (c2 + xc)
        t = jnp.maximum(sq, _EPS * _EPS)
        u = jnp.exp(_PHALF * jnp.log(t))
        o_ref[pl.ds(i * _CH, _CH), :] = u / jnp.sum(u, axis=1, keepdims=True)
        return carry

    jax.lax.fori_loop(0, _BN // _CH, step, 0, unroll=True)


def kernel(x, centers):
    N, F = x.shape
    K = centers.shape[0]
    cs = centers * (-2.0)
    c2 = jnp.sum(centers * centers, axis=1)[None, :]
    return pl.pallas_call(
        _fcm_body,
        grid=(N // _BN,),
        in_specs=[
            pl.BlockSpec((_BN, F), lambda i: (i, 0)),
            pl.BlockSpec((K, F), lambda i: (0, 0)),
            pl.BlockSpec((1, K), lambda i: (0, 0)),
        ],
        out_specs=pl.BlockSpec((_BN, K), lambda i: (i, 0)),
        out_shape=jax.ShapeDtypeStruct((N, K), jnp.float32),
        compiler_params=pltpu.CompilerParams(
            dimension_semantics=("parallel",)),
    )(x, cs, c2)


# speedup vs baseline: 1.3104x; 1.0013x over previous
import jax
import jax.numpy as jnp
from jax.experimental import pallas as pl
from jax.experimental.pallas import tpu as pltpu

_EPS = 1e-12
_M = 1.7
_PHALF = -1.0 / (_M - 1.0)

_BN = 16384
_CH = 2048


def _fcm_body(x_ref, cs_ref, c2_ref, o_ref):
    cs = cs_ref[...]
    c2 = c2_ref[...]

    def step(i, carry):
        x = x_ref[pl.ds(i * _CH, _CH), :]
        x2 = jnp.sum(x * x, axis=1, keepdims=True)
        xc = jax.lax.dot_general(
            x, cs, (((1,), (1,)), ((), ())),
            preferred_element_type=jnp.float32)
        sq = x2 + (c2 + xc)
        t = jnp.maximum(sq, _EPS * _EPS)
        u = jnp.exp(_PHALF * jnp.log(t))
        o_ref[pl.ds(i * _CH, _CH), :] = u / jnp.sum(u, axis=1, keepdims=True)
        return carry

    jax.lax.fori_loop(0, _BN // _CH, step, 0, unroll=True)


def kernel(x, centers):
    N, F = x.shape
    K = centers.shape[0]
    cs = centers * (-2.0)
    c2 = jnp.sum(centers * centers, axis=1)[None, :]
    return pl.pallas_call(
        _fcm_body,
        grid=(N // _BN,),
        in_specs=[
            pl.BlockSpec((_BN, F), lambda i: (i, 0)),
            pl.BlockSpec((K, F), lambda i: (0, 0)),
            pl.BlockSpec((1, K), lambda i: (0, 0)),
        ],
        out_specs=pl.BlockSpec((_BN, K), lambda i: (i, 0)),
        out_shape=jax.ShapeDtypeStruct((N, K), jnp.float32),
        compiler_params=pltpu.CompilerParams(
            dimension_semantics=("parallel",)),
    )(x, cs, c2)


# speedup vs baseline: 1.3267x; 1.0124x over previous
import jax
import jax.numpy as jnp
from jax.experimental import pallas as pl
from jax.experimental.pallas import tpu as pltpu

_EPS = 1e-12
_M = 1.7
_PHALF = -1.0 / (_M - 1.0)

_BN = 16384
_CH = 8192


def _fcm_body(x_ref, cs_ref, c2_ref, o_ref):
    cs = cs_ref[...]
    c2 = c2_ref[...]

    def step(i, carry):
        x = x_ref[pl.ds(i * _CH, _CH), :]
        x2 = jnp.sum(x * x, axis=1, keepdims=True)
        xc = jax.lax.dot_general(
            x, cs, (((1,), (1,)), ((), ())),
            preferred_element_type=jnp.float32)
        sq = x2 + (c2 + xc)
        t = jnp.maximum(sq, _EPS * _EPS)
        u = jnp.exp(_PHALF * jnp.log(t))
        o_ref[pl.ds(i * _CH, _CH), :] = u / jnp.sum(u, axis=1, keepdims=True)
        return carry

    jax.lax.fori_loop(0, _BN // _CH, step, 0, unroll=True)


def kernel(x, centers):
    N, F = x.shape
    K = centers.shape[0]
    cs = centers * (-2.0)
    c2 = jnp.sum(centers * centers, axis=1)[None, :]
    return pl.pallas_call(
        _fcm_body,
        grid=(N // _BN,),
        in_specs=[
            pl.BlockSpec((_BN, F), lambda i: (i, 0)),
            pl.BlockSpec((K, F), lambda i: (0, 0)),
            pl.BlockSpec((1, K), lambda i: (0, 0)),
        ],
        out_specs=pl.BlockSpec((_BN, K), lambda i: (i, 0)),
        out_shape=jax.ShapeDtypeStruct((N, K), jnp.float32),
        compiler_params=pltpu.CompilerParams(
            dimension_semantics=("parallel",)),
    )(x, cs, c2)
